# baseline (speedup 1.0000x reference)
import functools

import jax
import jax.numpy as jnp
from jax import lax
from jax.experimental import pallas as pl
from jax.experimental.pallas import tpu as pltpu
from jax.experimental.pallas import tpu_sc as plsc

_B, _H, _W = 2, 480, 360
_NVOX = _B * _H * _W
_NBINS = 256
_BINSZ = 1352
_NPASS = 8
_NW = 32
_CAP = 6272
_NPTS_PAD = _NW * _CAP
_CHUNK = 128

_LANES = 16
_NEG_INF = float("-inf")


def _splat_i32(x):
    return jnp.full((_LANES,), x, dtype=jnp.int32)


def _sc_scalar(ref, idxs):
    g = plsc.load_gather(ref, [_splat_i32(i) for i in idxs])
    return jnp.max(g)


def _lane_iota():
    return lax.iota(jnp.int32, _LANES)


def _sc_bin_kernel(v_hbm, bins_hbm, pids_hbm, vs_hbm, starts_hbm,
                   v_vm, b_vm, pid_vm, vv_vm, st_vm, cnt_sm):
    wid = lax.axis_index("s") * 2 + lax.axis_index("c")
    base = pl.multiple_of(wid * _CAP, _CAP)
    pltpu.sync_copy(v_hbm.at[pl.ds(base, _CAP)], v_vm)
    pltpu.sync_copy(bins_hbm.at[pl.ds(base, _CAP)], b_vm)

    def _zero(b, _):
        cnt_sm[b] = 0
        return _
    lax.fori_loop(0, 272, _zero, None)

    def _count(i, _):
        b = _sc_scalar(b_vm, [i])
        cnt_sm[b] = cnt_sm[b] + 1
        return _
    lax.fori_loop(0, _CAP, _count, None)

    def _prefix(b, acc):
        c = cnt_sm[b]
        cnt_sm[b] = acc
        return acc + c
    lax.fori_loop(0, 258, _prefix, 0)

    lane0 = _lane_iota() == 0

    def _save(b, _):
        val = cnt_sm[b]
        plsc.store_scatter(st_vm, [_splat_i32(b)], _splat_i32(val), mask=lane0)
        return _
    lax.fori_loop(0, 272, _save, None)

    def _place(i, _):
        b = _sc_scalar(b_vm, [i])
        v = _sc_scalar(v_vm, [i])
        pos = cnt_sm[b]
        cnt_sm[b] = pos + 1
        plsc.store_scatter(pid_vm, [_splat_i32(pos)], _splat_i32(base + i),
                           mask=lane0)
        plsc.store_scatter(vv_vm, [_splat_i32(pos)], _splat_i32(v), mask=lane0)
        return _
    lax.fori_loop(0, _CAP, _place, None)

    pltpu.sync_copy(pid_vm, pids_hbm.at[wid])
    pltpu.sync_copy(vv_vm, vs_hbm.at[wid])
    pltpu.sync_copy(st_vm, starts_hbm.at[wid])


def _sc_scatter_kernel(pids_hbm, vs_hbm, starts_hbm, feat_hbm, grid_hbm,
                       st_vm, pid_st, v_st, rows_vm, slab_vm, sem):
    wid = lax.axis_index("s") * 2 + lax.axis_index("c")
    pltpu.sync_copy(starts_hbm, st_vm)
    iot = _lane_iota()
    lane0 = iot == 0
    neg = jnp.full((_LANES,), _NEG_INF, dtype=jnp.float32)

    def _pass(p, _):
        g = p * _NW + wid
        gbase = pl.multiple_of(g * _BINSZ, 8)

        def _init(r, _):
            for c in range(4):
                plsc.store_scatter(slab_vm, [_splat_i32(r), iot + 16 * c], neg)
            return _
        lax.fori_loop(0, _BINSZ, _init, None)

        def _producer(j, _):
            s = _sc_scalar(st_vm, [j, g])
            e = _sc_scalar(st_vm, [j, g + 1])
            s_al = (s >> 7) << 7
            nch = jnp.where(e > s, (e - s_al + 127) >> 7, 0)

            def _chunk(k, _):
                cb = pl.multiple_of(s_al + k * _CHUNK, _CHUNK)
                pltpu.sync_copy(pids_hbm.at[j, pl.ds(cb, _CHUNK)], pid_st)
                pltpu.sync_copy(vs_hbm.at[j, pl.ds(cb, _CHUNK)], v_st)
                for r in range(_CHUNK // _LANES):
                    q = iot + (cb + r * _LANES)
                    ok = (q >= s) & (q < e)
                    pv = pid_st[pl.ds(r * _LANES, _LANES)]
                    pid_st[pl.ds(r * _LANES, _LANES)] = jnp.where(ok, pv, 0)
                pltpu.async_copy(feat_hbm.at[pid_st], rows_vm, sem).wait()

                def _point(i, _):
                    q = cb + i
                    @pl.when((q >= s) & (q < e))
                    def _():
                        v = _sc_scalar(v_st, [i])
                        row = v - gbase
                        for c in range(4):
                            col = iot + 16 * c
                            cur = plsc.load_gather(
                                slab_vm, [_splat_i32(row), col])
                            new = plsc.load_gather(
                                rows_vm, [_splat_i32(i), col])
                            plsc.store_scatter(
                                slab_vm, [_splat_i32(row), col],
                                jnp.maximum(cur, new))
                    return _
                lax.fori_loop(0, _CHUNK, _point, None)
                return _
            lax.fori_loop(0, nch, _chunk, None)
            return _
        lax.fori_loop(0, _NW, _producer, None)

        pltpu.sync_copy(slab_vm, grid_hbm.at[pl.ds(gbase, _BINSZ)])
        return _
    lax.fori_loop(0, _NPASS, _pass, None)


def _sc_scatter_max(v_lin, feat):
    n = v_lin.shape[0]
    v_pad = jnp.pad(v_lin, (0, _NPTS_PAD - n))
    valid = jnp.arange(_NPTS_PAD, dtype=jnp.int32) < n
    bins = jnp.where(valid, v_pad // _BINSZ, _NBINS).astype(jnp.int32)
    v_pad = v_pad.astype(jnp.int32)

    mesh = plsc.VectorSubcoreMesh(core_axis_name="c", subcore_axis_name="s")
    cparams = pltpu.CompilerParams(
        needs_layout_passes=False, use_tc_tiling_on_sc=False)

    bin_call = functools.partial(
        pl.kernel,
        compiler_params=cparams,
        out_type=[
            jax.ShapeDtypeStruct((_NW, _CAP), jnp.int32),
            jax.ShapeDtypeStruct((_NW, _CAP), jnp.int32),
            jax.ShapeDtypeStruct((_NW, 272), jnp.int32),
        ],
        mesh=mesh,
        scratch_types=[
            pltpu.VMEM((_CAP,), jnp.int32),
            pltpu.VMEM((_CAP,), jnp.int32),
            pltpu.VMEM((_CAP,), jnp.int32),
            pltpu.VMEM((_CAP,), jnp.int32),
            pltpu.VMEM((272,), jnp.int32),
            pltpu.SMEM((272,), jnp.int32),
        ],
    )
    pids, vs, starts = bin_call(_sc_bin_kernel)(v_pad, bins)

    scat_call = functools.partial(
        pl.kernel,
        compiler_params=cparams,
        out_type=jax.ShapeDtypeStruct((_NBINS * _BINSZ, 64), jnp.float32),
        mesh=mesh,
        scratch_types=[
            pltpu.VMEM((_NW, 272), jnp.int32),
            pltpu.VMEM((_CHUNK,), jnp.int32),
            pltpu.VMEM((_CHUNK,), jnp.int32),
            pltpu.VMEM((_CHUNK, 64), jnp.float32),
            pltpu.VMEM((_BINSZ, 64), jnp.float32),
            pltpu.SemaphoreType.DMA,
        ],
    )
    return scat_call(_sc_scatter_kernel)(pids, vs, starts, feat)


def _tc_stats_kernel(x_ref, o_ref):
    x = x_ref[...]
    x = jnp.where(x == _NEG_INF, 0.0, x)
    s = jnp.sum(x, axis=0)
    ss = jnp.sum(x * x, axis=0)
    @pl.when(pl.program_id(0) == 0)
    def _():
        o_ref[...] = jnp.zeros_like(o_ref)
    o_ref[0, :] += s
    o_ref[1, :] += ss


def _tc_stats(grid3):
    rows = grid3.shape[0]
    tb = 8640
    return pl.pallas_call(
        _tc_stats_kernel,
        grid=(rows // tb,),
        in_specs=[pl.BlockSpec((tb, 64), lambda i: (i, 0))],
        out_specs=pl.BlockSpec((2, 64), lambda i: (0, 0)),
        out_shape=jax.ShapeDtypeStruct((2, 64), jnp.float32),
    )(grid3)


_TR = 20
_NI = _H // _TR


def _tc_conv_kernel(xt_ref, xc_ref, xb_ref, a1_ref, c1_ref, w_ref, b_ref,
                    y_ref, st_ref):
    i = pl.program_id(1)
    a1 = a1_ref[0]
    c1 = c1_ref[0]

    def norm(t):
        return jnp.where(t == _NEG_INF, c1, t * a1 + c1)

    top = norm(xt_ref[0, _TR - 1:_TR]) * jnp.where(i > 0, 1.0, 0.0)
    mid = norm(xc_ref[0])
    bot = norm(xb_ref[0, 0:1]) * jnp.where(i < _NI - 1, 1.0, 0.0)
    xw = jnp.concatenate([top, mid, bot], axis=0)
    zc = jnp.zeros((_TR + 2, 1, 64), jnp.float32)
    xw = jnp.concatenate([zc, xw, zc], axis=1)

    acc = jnp.zeros((_TR * 360, 128), jnp.float32)
    for dy in range(3):
        for dx in range(3):
            xs = xw[dy:dy + _TR, dx:dx + 360, :].reshape(_TR * 360, 64)
            acc += jnp.dot(xs, w_ref[dy * 3 + dx],
                           preferred_element_type=jnp.float32)
    acc += b_ref[0]
    acc = jnp.maximum(acc, 0.0)
    y_ref[0] = acc.reshape(_TR, 360, 128)

    @pl.when((pl.program_id(0) == 0) & (i == 0))
    def _():
        st_ref[...] = jnp.zeros_like(st_ref)
    st_ref[0, :] += jnp.sum(acc, axis=0)
    st_ref[1, :] += jnp.sum(acc * acc, axis=0)


def _tc_conv(grid4, a1, c1, wt, bias):
    return pl.pallas_call(
        _tc_conv_kernel,
        grid=(_B, _NI),
        in_specs=[
            pl.BlockSpec((1, _TR, 360, 64),
                         lambda b, i: (b, jnp.maximum(i - 1, 0), 0, 0)),
            pl.BlockSpec((1, _TR, 360, 64), lambda b, i: (b, i, 0, 0)),
            pl.BlockSpec((1, _TR, 360, 64),
                         lambda b, i: (b, jnp.minimum(i + 1, _NI - 1), 0, 0)),
            pl.BlockSpec((1, 64), lambda b, i: (0, 0)),
            pl.BlockSpec((1, 64), lambda b, i: (0, 0)),
            pl.BlockSpec((9, 64, 128), lambda b, i: (0, 0, 0)),
            pl.BlockSpec((1, 128), lambda b, i: (0, 0)),
        ],
        out_specs=[
            pl.BlockSpec((1, _TR, 360, 128), lambda b, i: (b, i, 0, 0)),
            pl.BlockSpec((2, 128), lambda b, i: (0, 0)),
        ],
        out_shape=[
            jax.ShapeDtypeStruct((_B, _H, 360, 128), jnp.float32),
            jax.ShapeDtypeStruct((2, 128), jnp.float32),
        ],
    )(grid4, grid4, grid4, a1.reshape(1, 64), c1.reshape(1, 64), wt,
      bias.reshape(1, 128))


_TR2 = 40
_NI2 = _H // _TR2


def _tc_bn2_kernel(y_ref, a2_ref, c2_ref, o_ref):
    t = y_ref[0] * a2_ref[0] + c2_ref[0]
    o_ref[0] = jnp.transpose(t, (2, 0, 1))


def _tc_bn2(y4, a2, c2):
    return pl.pallas_call(
        _tc_bn2_kernel,
        grid=(_B, _NI2),
        in_specs=[
            pl.BlockSpec((1, _TR2, 360, 128), lambda b, i: (b, i, 0, 0)),
            pl.BlockSpec((1, 128), lambda b, i: (0, 0)),
            pl.BlockSpec((1, 128), lambda b, i: (0, 0)),
        ],
        out_specs=pl.BlockSpec((1, 128, _TR2, 360), lambda b, i: (b, 0, i, 0)),
        out_shape=jax.ShapeDtypeStruct((_B, 128, _H, 360), jnp.float32),
    )(y4, a2.reshape(1, 128), c2.reshape(1, 128))


def kernel(cylinder_features, cylinder_indices, polar_fea, conv_w, conv_b,
           bn1_gamma, bn1_beta, bn2_gamma, bn2_beta):
    eps = 1e-5
    n_elem = float(_NVOX)

    ib = cylinder_indices[:, 0]
    iy = cylinder_indices[:, 1]
    ix = cylinder_indices[:, 2]
    v_lin = (ib * _H + iy) * _W + ix

    grid = _sc_scatter_max(v_lin, cylinder_features)
    grid3 = grid[:_NVOX]

    st1 = _tc_stats(grid3)
    mu1 = st1[0] / n_elem
    var1 = st1[1] / n_elem - mu1 * mu1
    a1 = bn1_gamma * lax.rsqrt(var1 + eps)
    c1 = bn1_beta - mu1 * a1

    wt = jnp.transpose(conv_w, (2, 3, 1, 0)).reshape(9, 64, 128)
    grid4 = grid3.reshape(_B, _H, _W, 64)
    y4, st2 = _tc_conv(grid4, a1, c1, wt, conv_b)

    mu2 = st2[0] / n_elem
    var2 = st2[1] / n_elem - mu2 * mu2
    a2 = bn2_gamma * lax.rsqrt(var2 + eps)
    c2 = bn2_beta - mu2 * a2

    h = _tc_bn2(y4, a2, c2)
    return (h, polar_fea)


# speedup vs baseline: 1.0025x; 1.0025x over previous
import functools

import jax
import jax.numpy as jnp
from jax import lax
from jax.experimental import pallas as pl
from jax.experimental.pallas import tpu as pltpu
from jax.experimental.pallas import tpu_sc as plsc

_B, _H, _W = 2, 480, 360
_NVOX = _B * _H * _W
_NBINS = 256
_BINSZ = 1352
_NPASS = 8
_NW = 32
_CAP = 6272
_NPTS_PAD = _NW * _CAP
_CHUNK = 128

_LANES = 16
_NEG_INF = float("-inf")


def _splat_i32(x):
    return jnp.full((_LANES,), x, dtype=jnp.int32)


def _sc_scalar(ref, idxs):
    g = plsc.load_gather(ref, [_splat_i32(i) for i in idxs])
    return jnp.max(g)


def _lane_iota():
    return lax.iota(jnp.int32, _LANES)


def _sc_bin_kernel(v_hbm, bins_hbm, pids_hbm, vs_hbm, starts_hbm,
                   v_vm, b_vm, pid_vm, vv_vm, st_vm, cnt_sm):
    wid = lax.axis_index("s") * 2 + lax.axis_index("c")
    base = pl.multiple_of(wid * _CAP, _CAP)
    pltpu.sync_copy(v_hbm.at[pl.ds(base, _CAP)], v_vm)
    pltpu.sync_copy(bins_hbm.at[pl.ds(base, _CAP)], b_vm)

    def _zero(b, _):
        cnt_sm[b] = 0
        return _
    lax.fori_loop(0, 272, _zero, None)

    def _count(i, _):
        b = _sc_scalar(b_vm, [i])
        cnt_sm[b] = cnt_sm[b] + 1
        return _
    lax.fori_loop(0, _CAP, _count, None)

    def _prefix(b, acc):
        c = cnt_sm[b]
        cnt_sm[b] = acc
        return acc + c
    lax.fori_loop(0, 258, _prefix, 0)

    lane0 = _lane_iota() == 0

    def _save(b, _):
        val = cnt_sm[b]
        plsc.store_scatter(st_vm, [_splat_i32(b)], _splat_i32(val), mask=lane0)
        return _
    lax.fori_loop(0, 272, _save, None)

    def _place(i, _):
        b = _sc_scalar(b_vm, [i])
        v = _sc_scalar(v_vm, [i])
        pos = cnt_sm[b]
        cnt_sm[b] = pos + 1
        plsc.store_scatter(pid_vm, [_splat_i32(pos)], _splat_i32(base + i),
                           mask=lane0)
        plsc.store_scatter(vv_vm, [_splat_i32(pos)], _splat_i32(v), mask=lane0)
        return _
    lax.fori_loop(0, _CAP, _place, None)

    pltpu.sync_copy(pid_vm, pids_hbm.at[wid])
    pltpu.sync_copy(vv_vm, vs_hbm.at[wid])
    pltpu.sync_copy(st_vm, starts_hbm.at[wid])


def _lane_at(vec, lane):
    return jnp.max(jnp.where(_lane_iota() == lane, vec, 0))


def _sc_scatter_kernel(pids_hbm, vs_hbm, starts_hbm, feat_hbm, grid_hbm,
                       st_vm, pid_st, v_st, rows_vm, slab_vm, sem):
    wid = lax.axis_index("s") * 2 + lax.axis_index("c")
    pltpu.sync_copy(starts_hbm, st_vm)
    iot = _lane_iota()
    neg = jnp.full((_LANES,), _NEG_INF, dtype=jnp.float32)

    def _elem(j, idx):
        base = pl.multiple_of((idx >> 4) << 4, 16)
        return _lane_at(st_vm[j, pl.ds(base, 16)], idx & 15)

    def _pass(p, _):
        g = p * _NW + wid
        gbase = pl.multiple_of(g * _BINSZ, 8)

        def _init(r, _):
            for c in range(4):
                slab_vm[r, pl.ds(16 * c, 16)] = neg
            return _
        lax.fori_loop(0, _BINSZ, _init, None)

        def _producer(j, _):
            s = _elem(j, g)
            e = _elem(j, g + 1)
            s_al = (s >> 7) << 7
            nch = jnp.where(e > s, (e - s_al + 127) >> 7, 0)

            def _chunk(k, _):
                cb = pl.multiple_of(s_al + k * _CHUNK, _CHUNK)
                pltpu.sync_copy(pids_hbm.at[j, pl.ds(cb, _CHUNK)], pid_st)
                pltpu.sync_copy(vs_hbm.at[j, pl.ds(cb, _CHUNK)], v_st)
                for r in range(_CHUNK // _LANES):
                    q = iot + (cb + r * _LANES)
                    ok = (q >= s) & (q < e)
                    pv = pid_st[pl.ds(r * _LANES, _LANES)]
                    pid_st[pl.ds(r * _LANES, _LANES)] = jnp.where(ok, pv, 0)
                pltpu.async_copy(feat_hbm.at[pid_st], rows_vm, sem).wait()

                lo = jnp.maximum(s - cb, 0)
                hi = jnp.minimum(e - cb, _CHUNK)

                def _point(i, _):
                    vbase = pl.multiple_of((i >> 4) << 4, 16)
                    v = _lane_at(v_st[pl.ds(vbase, 16)], i & 15)
                    row = v - gbase
                    for c in range(4):
                        cur = slab_vm[row, pl.ds(16 * c, 16)]
                        new = rows_vm[i, pl.ds(16 * c, 16)]
                        slab_vm[row, pl.ds(16 * c, 16)] = jnp.maximum(cur, new)
                    return _
                lax.fori_loop(lo, hi, _point, None)
                return _
            lax.fori_loop(0, nch, _chunk, None)
            return _
        lax.fori_loop(0, _NW, _producer, None)

        pltpu.sync_copy(slab_vm, grid_hbm.at[pl.ds(gbase, _BINSZ)])
        return _
    lax.fori_loop(0, _NPASS, _pass, None)


def _sc_scatter_max(v_lin, feat):
    n = v_lin.shape[0]
    v_pad = jnp.pad(v_lin, (0, _NPTS_PAD - n))
    valid = jnp.arange(_NPTS_PAD, dtype=jnp.int32) < n
    bins = jnp.where(valid, v_pad // _BINSZ, _NBINS).astype(jnp.int32)
    v_pad = v_pad.astype(jnp.int32)

    mesh = plsc.VectorSubcoreMesh(core_axis_name="c", subcore_axis_name="s")
    cparams = pltpu.CompilerParams(
        needs_layout_passes=False, use_tc_tiling_on_sc=False)

    bin_call = functools.partial(
        pl.kernel,
        compiler_params=cparams,
        out_type=[
            jax.ShapeDtypeStruct((_NW, _CAP), jnp.int32),
            jax.ShapeDtypeStruct((_NW, _CAP), jnp.int32),
            jax.ShapeDtypeStruct((_NW, 272), jnp.int32),
        ],
        mesh=mesh,
        scratch_types=[
            pltpu.VMEM((_CAP,), jnp.int32),
            pltpu.VMEM((_CAP,), jnp.int32),
            pltpu.VMEM((_CAP,), jnp.int32),
            pltpu.VMEM((_CAP,), jnp.int32),
            pltpu.VMEM((272,), jnp.int32),
            pltpu.SMEM((272,), jnp.int32),
        ],
    )
    pids, vs, starts = bin_call(_sc_bin_kernel)(v_pad, bins)

    scat_call = functools.partial(
        pl.kernel,
        compiler_params=cparams,
        out_type=jax.ShapeDtypeStruct((_NBINS * _BINSZ, 64), jnp.float32),
        mesh=mesh,
        scratch_types=[
            pltpu.VMEM((_NW, 272), jnp.int32),
            pltpu.VMEM((_CHUNK,), jnp.int32),
            pltpu.VMEM((_CHUNK,), jnp.int32),
            pltpu.VMEM((_CHUNK, 64), jnp.float32),
            pltpu.VMEM((_BINSZ, 64), jnp.float32),
            pltpu.SemaphoreType.DMA,
        ],
    )
    return scat_call(_sc_scatter_kernel)(pids, vs, starts, feat)


def _tc_stats_kernel(x_ref, o_ref):
    x = x_ref[...]
    x = jnp.where(x == _NEG_INF, 0.0, x)
    s = jnp.sum(x, axis=0)
    ss = jnp.sum(x * x, axis=0)
    @pl.when(pl.program_id(0) == 0)
    def _():
        o_ref[...] = jnp.zeros_like(o_ref)
    o_ref[0, :] += s
    o_ref[1, :] += ss


def _tc_stats(grid3):
    rows = grid3.shape[0]
    tb = 8640
    return pl.pallas_call(
        _tc_stats_kernel,
        grid=(rows // tb,),
        in_specs=[pl.BlockSpec((tb, 64), lambda i: (i, 0))],
        out_specs=pl.BlockSpec((2, 64), lambda i: (0, 0)),
        out_shape=jax.ShapeDtypeStruct((2, 64), jnp.float32),
    )(grid3)


_TR = 20
_NI = _H // _TR


def _tc_conv_kernel(xt_ref, xc_ref, xb_ref, a1_ref, c1_ref, w_ref, b_ref,
                    y_ref, st_ref):
    i = pl.program_id(1)
    a1 = a1_ref[0]
    c1 = c1_ref[0]

    def norm(t):
        return jnp.where(t == _NEG_INF, c1, t * a1 + c1)

    top = norm(xt_ref[0, _TR - 1:_TR]) * jnp.where(i > 0, 1.0, 0.0)
    mid = norm(xc_ref[0])
    bot = norm(xb_ref[0, 0:1]) * jnp.where(i < _NI - 1, 1.0, 0.0)
    xw = jnp.concatenate([top, mid, bot], axis=0)
    zc = jnp.zeros((_TR + 2, 1, 64), jnp.float32)
    xw = jnp.concatenate([zc, xw, zc], axis=1)

    acc = jnp.zeros((_TR * 360, 128), jnp.float32)
    for dy in range(3):
        for dx in range(3):
            xs = xw[dy:dy + _TR, dx:dx + 360, :].reshape(_TR * 360, 64)
            acc += jnp.dot(xs, w_ref[dy * 3 + dx],
                           preferred_element_type=jnp.float32)
    acc += b_ref[0]
    acc = jnp.maximum(acc, 0.0)
    y_ref[0] = acc.reshape(_TR, 360, 128)

    @pl.when((pl.program_id(0) == 0) & (i == 0))
    def _():
        st_ref[...] = jnp.zeros_like(st_ref)
    st_ref[0, :] += jnp.sum(acc, axis=0)
    st_ref[1, :] += jnp.sum(acc * acc, axis=0)


def _tc_conv(grid4, a1, c1, wt, bias):
    return pl.pallas_call(
        _tc_conv_kernel,
        grid=(_B, _NI),
        in_specs=[
            pl.BlockSpec((1, _TR, 360, 64),
                         lambda b, i: (b, jnp.maximum(i - 1, 0), 0, 0)),
            pl.BlockSpec((1, _TR, 360, 64), lambda b, i: (b, i, 0, 0)),
            pl.BlockSpec((1, _TR, 360, 64),
                         lambda b, i: (b, jnp.minimum(i + 1, _NI - 1), 0, 0)),
            pl.BlockSpec((1, 64), lambda b, i: (0, 0)),
            pl.BlockSpec((1, 64), lambda b, i: (0, 0)),
            pl.BlockSpec((9, 64, 128), lambda b, i: (0, 0, 0)),
            pl.BlockSpec((1, 128), lambda b, i: (0, 0)),
        ],
        out_specs=[
            pl.BlockSpec((1, _TR, 360, 128), lambda b, i: (b, i, 0, 0)),
            pl.BlockSpec((2, 128), lambda b, i: (0, 0)),
        ],
        out_shape=[
            jax.ShapeDtypeStruct((_B, _H, 360, 128), jnp.float32),
            jax.ShapeDtypeStruct((2, 128), jnp.float32),
        ],
    )(grid4, grid4, grid4, a1.reshape(1, 64), c1.reshape(1, 64), wt,
      bias.reshape(1, 128))


_TR2 = 40
_NI2 = _H // _TR2


def _tc_bn2_kernel(y_ref, a2_ref, c2_ref, o_ref):
    t = y_ref[0] * a2_ref[0] + c2_ref[0]
    o_ref[0] = jnp.transpose(t, (2, 0, 1))


def _tc_bn2(y4, a2, c2):
    return pl.pallas_call(
        _tc_bn2_kernel,
        grid=(_B, _NI2),
        in_specs=[
            pl.BlockSpec((1, _TR2, 360, 128), lambda b, i: (b, i, 0, 0)),
            pl.BlockSpec((1, 128), lambda b, i: (0, 0)),
            pl.BlockSpec((1, 128), lambda b, i: (0, 0)),
        ],
        out_specs=pl.BlockSpec((1, 128, _TR2, 360), lambda b, i: (b, 0, i, 0)),
        out_shape=jax.ShapeDtypeStruct((_B, 128, _H, 360), jnp.float32),
    )(y4, a2.reshape(1, 128), c2.reshape(1, 128))


def kernel(cylinder_features, cylinder_indices, polar_fea, conv_w, conv_b,
           bn1_gamma, bn1_beta, bn2_gamma, bn2_beta):
    eps = 1e-5
    n_elem = float(_NVOX)

    ib = cylinder_indices[:, 0]
    iy = cylinder_indices[:, 1]
    ix = cylinder_indices[:, 2]
    v_lin = (ib * _H + iy) * _W + ix

    grid = _sc_scatter_max(v_lin, cylinder_features)
    grid3 = grid[:_NVOX]

    st1 = _tc_stats(grid3)
    mu1 = st1[0] / n_elem
    var1 = st1[1] / n_elem - mu1 * mu1
    a1 = bn1_gamma * lax.rsqrt(var1 + eps)
    c1 = bn1_beta - mu1 * a1

    wt = jnp.transpose(conv_w, (2, 3, 1, 0)).reshape(9, 64, 128)
    grid4 = grid3.reshape(_B, _H, _W, 64)
    y4, st2 = _tc_conv(grid4, a1, c1, wt, conv_b)

    mu2 = st2[0] / n_elem
    var2 = st2[1] / n_elem - mu2 * mu2
    a2 = bn2_gamma * lax.rsqrt(var2 + eps)
    c2 = bn2_beta - mu2 * a2

    h = _tc_bn2(y4, a2, c2)
    return (h, polar_fea)


# speedup vs baseline: 1.0025x; 1.0000x over previous
import functools

import jax
import jax.numpy as jnp
from jax import lax
from jax.experimental import pallas as pl
from jax.experimental.pallas import tpu as pltpu
from jax.experimental.pallas import tpu_sc as plsc

_B, _H, _W = 2, 480, 360
_NVOX = _B * _H * _W
_NBINS = 256
_BINSZ = 1352
_NPASS = 8
_NW = 32
_CAP = 6272
_NPTS_PAD = _NW * _CAP
_CHUNK = 128

_LANES = 16
_NEG_INF = float("-inf")


def _splat_i32(x):
    return jnp.full((_LANES,), x, dtype=jnp.int32)


def _sc_scalar(ref, idxs):
    g = plsc.load_gather(ref, [_splat_i32(i) for i in idxs])
    return jnp.max(g)


def _lane_iota():
    return lax.iota(jnp.int32, _LANES)


def _sc_bin_kernel(v_hbm, bins_hbm, pids_hbm, vs_hbm, starts_hbm,
                   v_vm, b_vm, pid_vm, vv_vm, st_vm, cnt_sm):
    wid = lax.axis_index("s") * 2 + lax.axis_index("c")
    base = pl.multiple_of(wid * _CAP, _CAP)
    pltpu.sync_copy(v_hbm.at[pl.ds(base, _CAP)], v_vm)
    pltpu.sync_copy(bins_hbm.at[pl.ds(base, _CAP)], b_vm)

    def _zero(b, _):
        cnt_sm[b] = 0
        return _
    lax.fori_loop(0, 272, _zero, None)

    def _count(i, _):
        b = _sc_scalar(b_vm, [i])
        cnt_sm[b] = cnt_sm[b] + 1
        return _
    lax.fori_loop(0, _CAP, _count, None)

    def _prefix(b, acc):
        c = cnt_sm[b]
        cnt_sm[b] = acc
        return acc + c
    lax.fori_loop(0, 258, _prefix, 0)

    lane0 = _lane_iota() == 0

    def _save(b, _):
        val = cnt_sm[b]
        plsc.store_scatter(st_vm, [_splat_i32(b)], _splat_i32(val), mask=lane0)
        return _
    lax.fori_loop(0, 272, _save, None)

    def _place(i, _):
        b = _sc_scalar(b_vm, [i])
        v = _sc_scalar(v_vm, [i])
        pos = cnt_sm[b]
        cnt_sm[b] = pos + 1
        plsc.store_scatter(pid_vm, [_splat_i32(pos)], _splat_i32(base + i),
                           mask=lane0)
        plsc.store_scatter(vv_vm, [_splat_i32(pos)], _splat_i32(v), mask=lane0)
        return _
    lax.fori_loop(0, _CAP, _place, None)

    pltpu.sync_copy(pid_vm, pids_hbm.at[wid])
    pltpu.sync_copy(vv_vm, vs_hbm.at[wid])
    pltpu.sync_copy(st_vm, starts_hbm.at[wid])


def _lane_at(vec, lane):
    return jnp.max(jnp.where(_lane_iota() == lane, vec, 0))


def _sc_scatter_kernel(pids_hbm, vs_hbm, starts_hbm, feat_hbm, grid_hbm,
                       st_vm, pid_st, v_st, rows_vm, slab_vm, sem):
    wid = lax.axis_index("s") * 2 + lax.axis_index("c")
    pltpu.sync_copy(starts_hbm, st_vm)
    iot = _lane_iota()
    neg = jnp.full((_LANES,), _NEG_INF, dtype=jnp.float32)

    def _elem(j, idx):
        base = pl.multiple_of((idx >> 4) << 4, 16)
        return _lane_at(st_vm[j, pl.ds(base, 16)], idx & 15)

    def _pass(p, _):
        g = p * _NW + wid
        gbase = pl.multiple_of(g * _BINSZ, 8)

        def _init(r, _):
            for c in range(4):
                slab_vm[r, pl.ds(16 * c, 16)] = neg
            return _
        lax.fori_loop(0, _BINSZ, _init, None)

        def _producer(j, _):
            s = _elem(j, g)
            e = _elem(j, g + 1)
            s_al = (s >> 7) << 7
            nch = jnp.where(e > s, (e - s_al + 127) >> 7, 0)

            def _chunk(k, _):
                cb = pl.multiple_of(s_al + k * _CHUNK, _CHUNK)
                cp1 = pltpu.async_copy(
                    pids_hbm.at[j, pl.ds(cb, _CHUNK)], pid_st, sem)
                cp2 = pltpu.async_copy(
                    vs_hbm.at[j, pl.ds(cb, _CHUNK)], v_st, sem)
                cp1.wait()
                cp2.wait()
                for r in range(_CHUNK // _LANES):
                    q = iot + (cb + r * _LANES)
                    ok = (q >= s) & (q < e)
                    pv = pid_st[pl.ds(r * _LANES, _LANES)]
                    pid_st[pl.ds(r * _LANES, _LANES)] = jnp.where(ok, pv, 0)
                pltpu.async_copy(feat_hbm.at[pid_st], rows_vm, sem).wait()

                lo = jnp.maximum(s - cb, 0)
                hi = jnp.minimum(e - cb, _CHUNK)

                def _point(i, _):
                    vbase = pl.multiple_of((i >> 4) << 4, 16)
                    v = _lane_at(v_st[pl.ds(vbase, 16)], i & 15)
                    row = v - gbase
                    for c in range(4):
                        cur = slab_vm[row, pl.ds(16 * c, 16)]
                        new = rows_vm[i, pl.ds(16 * c, 16)]
                        slab_vm[row, pl.ds(16 * c, 16)] = jnp.maximum(cur, new)
                    return _
                lax.fori_loop(lo, hi, _point, None)
                return _
            lax.fori_loop(0, nch, _chunk, None)
            return _
        lax.fori_loop(0, _NW, _producer, None)

        pltpu.sync_copy(slab_vm, grid_hbm.at[pl.ds(gbase, _BINSZ)])
        return _
    lax.fori_loop(0, _NPASS, _pass, None)


def _sc_scatter_max(v_lin, feat):
    n = v_lin.shape[0]
    v_pad = jnp.pad(v_lin, (0, _NPTS_PAD - n))
    valid = jnp.arange(_NPTS_PAD, dtype=jnp.int32) < n
    bins = jnp.where(valid, v_pad // _BINSZ, _NBINS).astype(jnp.int32)
    v_pad = v_pad.astype(jnp.int32)

    mesh = plsc.VectorSubcoreMesh(core_axis_name="c", subcore_axis_name="s")
    cparams = pltpu.CompilerParams(
        needs_layout_passes=False, use_tc_tiling_on_sc=False)

    bin_call = functools.partial(
        pl.kernel,
        compiler_params=cparams,
        out_type=[
            jax.ShapeDtypeStruct((_NW, _CAP), jnp.int32),
            jax.ShapeDtypeStruct((_NW, _CAP), jnp.int32),
            jax.ShapeDtypeStruct((_NW, 272), jnp.int32),
        ],
        mesh=mesh,
        scratch_types=[
            pltpu.VMEM((_CAP,), jnp.int32),
            pltpu.VMEM((_CAP,), jnp.int32),
            pltpu.VMEM((_CAP,), jnp.int32),
            pltpu.VMEM((_CAP,), jnp.int32),
            pltpu.VMEM((272,), jnp.int32),
            pltpu.SMEM((272,), jnp.int32),
        ],
    )
    pids, vs, starts = bin_call(_sc_bin_kernel)(v_pad, bins)

    scat_call = functools.partial(
        pl.kernel,
        compiler_params=cparams,
        out_type=jax.ShapeDtypeStruct((_NBINS * _BINSZ, 64), jnp.float32),
        mesh=mesh,
        scratch_types=[
            pltpu.VMEM((_NW, 272), jnp.int32),
            pltpu.VMEM((_CHUNK,), jnp.int32),
            pltpu.VMEM((_CHUNK,), jnp.int32),
            pltpu.VMEM((_CHUNK, 64), jnp.float32),
            pltpu.VMEM((_BINSZ, 64), jnp.float32),
            pltpu.SemaphoreType.DMA,
        ],
    )
    return scat_call(_sc_scatter_kernel)(pids, vs, starts, feat)


def _tc_stats_kernel(x_ref, o_ref):
    x = x_ref[...]
    x = jnp.where(x == _NEG_INF, 0.0, x)
    s = jnp.sum(x, axis=0)
    ss = jnp.sum(x * x, axis=0)
    @pl.when(pl.program_id(0) == 0)
    def _():
        o_ref[...] = jnp.zeros_like(o_ref)
    o_ref[0, :] += s
    o_ref[1, :] += ss


def _tc_stats(grid3):
    rows = grid3.shape[0]
    tb = 8640
    return pl.pallas_call(
        _tc_stats_kernel,
        grid=(rows // tb,),
        in_specs=[pl.BlockSpec((tb, 64), lambda i: (i, 0))],
        out_specs=pl.BlockSpec((2, 64), lambda i: (0, 0)),
        out_shape=jax.ShapeDtypeStruct((2, 64), jnp.float32),
    )(grid3)


_TR = 20
_NI = _H // _TR


def _tc_conv_kernel(xt_ref, xc_ref, xb_ref, a1_ref, c1_ref, w_ref, b_ref,
                    y_ref, st_ref):
    i = pl.program_id(1)
    a1 = a1_ref[0]
    c1 = c1_ref[0]

    def norm(t):
        return jnp.where(t == _NEG_INF, c1, t * a1 + c1)

    top = norm(xt_ref[0, _TR - 1:_TR]) * jnp.where(i > 0, 1.0, 0.0)
    mid = norm(xc_ref[0])
    bot = norm(xb_ref[0, 0:1]) * jnp.where(i < _NI - 1, 1.0, 0.0)
    xw = jnp.concatenate([top, mid, bot], axis=0)
    zc = jnp.zeros((_TR + 2, 1, 64), jnp.float32)
    xw = jnp.concatenate([zc, xw, zc], axis=1)

    acc = jnp.zeros((_TR * 360, 128), jnp.float32)
    for dy in range(3):
        for dx in range(3):
            xs = xw[dy:dy + _TR, dx:dx + 360, :].reshape(_TR * 360, 64)
            acc += jnp.dot(xs, w_ref[dy * 3 + dx],
                           preferred_element_type=jnp.float32)
    acc += b_ref[0]
    acc = jnp.maximum(acc, 0.0)
    y_ref[0] = acc.reshape(_TR, 360, 128)

    @pl.when((pl.program_id(0) == 0) & (i == 0))
    def _():
        st_ref[...] = jnp.zeros_like(st_ref)
    st_ref[0, :] += jnp.sum(acc, axis=0)
    st_ref[1, :] += jnp.sum(acc * acc, axis=0)


def _tc_conv(grid4, a1, c1, wt, bias):
    return pl.pallas_call(
        _tc_conv_kernel,
        grid=(_B, _NI),
        in_specs=[
            pl.BlockSpec((1, _TR, 360, 64),
                         lambda b, i: (b, jnp.maximum(i - 1, 0), 0, 0)),
            pl.BlockSpec((1, _TR, 360, 64), lambda b, i: (b, i, 0, 0)),
            pl.BlockSpec((1, _TR, 360, 64),
                         lambda b, i: (b, jnp.minimum(i + 1, _NI - 1), 0, 0)),
            pl.BlockSpec((1, 64), lambda b, i: (0, 0)),
            pl.BlockSpec((1, 64), lambda b, i: (0, 0)),
            pl.BlockSpec((9, 64, 128), lambda b, i: (0, 0, 0)),
            pl.BlockSpec((1, 128), lambda b, i: (0, 0)),
        ],
        out_specs=[
            pl.BlockSpec((1, _TR, 360, 128), lambda b, i: (b, i, 0, 0)),
            pl.BlockSpec((2, 128), lambda b, i: (0, 0)),
        ],
        out_shape=[
            jax.ShapeDtypeStruct((_B, _H, 360, 128), jnp.float32),
            jax.ShapeDtypeStruct((2, 128), jnp.float32),
        ],
    )(grid4, grid4, grid4, a1.reshape(1, 64), c1.reshape(1, 64), wt,
      bias.reshape(1, 128))


_TR2 = 40
_NI2 = _H // _TR2


def _tc_bn2_kernel(y_ref, a2_ref, c2_ref, o_ref):
    t = y_ref[0] * a2_ref[0] + c2_ref[0]
    o_ref[0] = jnp.transpose(t, (2, 0, 1))


def _tc_bn2(y4, a2, c2):
    return pl.pallas_call(
        _tc_bn2_kernel,
        grid=(_B, _NI2),
        in_specs=[
            pl.BlockSpec((1, _TR2, 360, 128), lambda b, i: (b, i, 0, 0)),
            pl.BlockSpec((1, 128), lambda b, i: (0, 0)),
            pl.BlockSpec((1, 128), lambda b, i: (0, 0)),
        ],
        out_specs=pl.BlockSpec((1, 128, _TR2, 360), lambda b, i: (b, 0, i, 0)),
        out_shape=jax.ShapeDtypeStruct((_B, 128, _H, 360), jnp.float32),
    )(y4, a2.reshape(1, 128), c2.reshape(1, 128))


def kernel(cylinder_features, cylinder_indices, polar_fea, conv_w, conv_b,
           bn1_gamma, bn1_beta, bn2_gamma, bn2_beta):
    eps = 1e-5
    n_elem = float(_NVOX)

    ib = cylinder_indices[:, 0]
    iy = cylinder_indices[:, 1]
    ix = cylinder_indices[:, 2]
    v_lin = (ib * _H + iy) * _W + ix

    grid = _sc_scatter_max(v_lin, cylinder_features)
    grid3 = grid[:_NVOX]

    st1 = _tc_stats(grid3)
    mu1 = st1[0] / n_elem
    var1 = st1[1] / n_elem - mu1 * mu1
    a1 = bn1_gamma * lax.rsqrt(var1 + eps)
    c1 = bn1_beta - mu1 * a1

    wt = jnp.transpose(conv_w, (2, 3, 1, 0)).reshape(9, 64, 128)
    grid4 = grid3.reshape(_B, _H, _W, 64)
    y4, st2 = _tc_conv(grid4, a1, c1, wt, conv_b)

    mu2 = st2[0] / n_elem
    var2 = st2[1] / n_elem - mu2 * mu2
    a2 = bn2_gamma * lax.rsqrt(var2 + eps)
    c2 = bn2_beta - mu2 * a2

    h = _tc_bn2(y4, a2, c2)
    return (h, polar_fea)
